# baseline (speedup 1.0000x reference)
import functools

import jax
import jax.numpy as jnp
from jax import lax
from jax.experimental import pallas as pl
from jax.experimental.pallas import tpu as pltpu
from jax.experimental.pallas import tpu_sc as plsc

C = 128
O = 128
OW = O // 2
N = 10000
K = 16
L = 16

NC, NS = 2, 16
NW = NC * NS
NODES_W = 320
N_PAD = NW * NODES_W
CH = 4
NCH = NODES_W // CH
BN = 2560


def _mm_body(x_ref, wd_ref, w2_ref, hb_ref, y1_ref, y2_ref):
    xb = x_ref[...]
    hb = hb_ref[0:1, :]
    dn = (((0,), (1,)), ((), ()))
    y1_ref[...] = (lax.dot_general(xb, wd_ref[...], dn,
                                   preferred_element_type=jnp.float32)
                   + hb).astype(jnp.bfloat16)
    y2_ref[...] = (lax.dot_general(xb, w2_ref[...], dn,
                                   preferred_element_type=jnp.float32)
                   + hb).astype(jnp.bfloat16)


def _build_tables(xp, wd, w2, hb):
    nb = N_PAD // BN
    return pl.pallas_call(
        _mm_body,
        grid=(nb,),
        in_specs=[
            pl.BlockSpec((C, BN), lambda i: (0, i)),
            pl.BlockSpec((O, C), lambda i: (0, 0)),
            pl.BlockSpec((O, C), lambda i: (0, 0)),
            pl.BlockSpec((8, O), lambda i: (0, 0)),
        ],
        out_specs=[
            pl.BlockSpec((BN, O), lambda i: (i, 0)),
            pl.BlockSpec((BN, O), lambda i: (i, 0)),
        ],
        out_shape=[
            jax.ShapeDtypeStruct((N_PAD, O), jnp.bfloat16),
            jax.ShapeDtypeStruct((N_PAD, O), jnp.bfloat16),
        ],
    )(xp, wd, w2, hb)


@functools.partial(
    pl.kernel,
    mesh=plsc.VectorSubcoreMesh(core_axis_name="c", subcore_axis_name="s"),
    out_type=jax.ShapeDtypeStruct((N_PAD, OW), jnp.int32),
    compiler_params=pltpu.CompilerParams(use_tc_tiling_on_sc=False,
                                        needs_layout_passes=False),
    scratch_types=[
        pltpu.VMEM((NCH, CH * K), jnp.int32),
        pltpu.VMEM((NCH, CH * K), jnp.int32),
        pltpu.VMEM((4, CH * K, OW), jnp.int32),
        pltpu.VMEM((4, CH * K, OW), jnp.int32),
        pltpu.VMEM((4, CH, OW), jnp.int32),
        pltpu.VMEM_SHARED((N_PAD, OW), jnp.int32),
        pltpu.VMEM_SHARED((N_PAD, OW), jnp.int32),
        pltpu.SemaphoreType.DMA,
        pltpu.SemaphoreType.DMA,
        pltpu.SemaphoreType.DMA,
        pltpu.SemaphoreType.DMA,
        pltpu.SemaphoreType.DMA,
        pltpu.SemaphoreType.DMA,
        pltpu.SemaphoreType.DMA,
        pltpu.SemaphoreType.DMA,
    ],
)
def _sc_gather_max(y1_hbm, y2_hbm, i1_hbm, i0_hbm, out_hbm,
                   i1_v, i0_v, r1_v, r2_v, o_v, sh1, sh2,
                   sg0, sg1, sg2, sg3, so0, so1, so2, so3):
    wid = lax.axis_index("s") * NC + lax.axis_index("c")
    nbase = wid * NODES_W
    pltpu.sync_copy(i1_hbm.at[wid], i1_v)
    pltpu.sync_copy(i0_hbm.at[wid], i0_v)

    sid = lax.axis_index("s")
    rpt = N_PAD // NS
    seg = pl.ds(sid * rpt, rpt)
    pltpu.sync_copy(y1_hbm.at[seg], sh1.at[seg])
    pltpu.sync_copy(y2_hbm.at[seg], sh2.at[seg])
    plsc.subcore_barrier()

    sgs = (sg0, sg1, sg2, sg3)
    sos = (so0, so1, so2, so3)

    def issue_gathers(c, b):
        pltpu.async_copy(sh1.at[i1_v.at[c]], r1_v.at[b], sgs[b])
        pltpu.async_copy(sh2.at[i0_v.at[c]], r2_v.at[b], sgs[b])

    def wait_gathers(c, b):
        pltpu.make_async_copy(sh1.at[i1_v.at[c]], r1_v.at[b], sgs[b]).wait()
        pltpu.make_async_copy(sh2.at[i0_v.at[c]], r2_v.at[b], sgs[b]).wait()

    def out_slice(c):
        return out_hbm.at[pl.ds(nbase + c * CH, CH)]

    def compute(b):
        def node_body(n, carry):
            row = n * K
            for j in range(OW // L):
                sl = pl.ds(j * L, L)
                acc = (plsc.bitcast(r1_v[b, row, sl], jnp.bfloat16)
                       + plsc.bitcast(r2_v[b, row, sl], jnp.bfloat16))
                for k in range(1, K):
                    acc = jnp.maximum(
                        acc,
                        plsc.bitcast(r1_v[b, row + k, sl], jnp.bfloat16)
                        + plsc.bitcast(r2_v[b, row + k, sl], jnp.bfloat16))
                o_v[b, n, sl] = plsc.bitcast(acc, jnp.int32)
            return carry

        lax.fori_loop(0, CH, node_body, 0, unroll=False)

    NBUF = 4
    for b0 in range(NBUF):
        issue_gathers(b0, b0)

    def group_body(g, carry):
        for b in range(NBUF):
            c = NBUF * g + b
            wait_gathers(c, b)

            @pl.when(g > 0)
            def _():
                pltpu.make_async_copy(o_v.at[b], out_slice(c - NBUF), sos[b]).wait()

            compute(b)
            pltpu.async_copy(o_v.at[b], out_slice(c), sos[b])

            @pl.when(g < NCH // NBUF - 1)
            def _():
                issue_gathers(c + NBUF, b)
        return carry

    lax.fori_loop(0, NCH // 4, group_body, 0, unroll=False)
    for b0 in range(NBUF):
        pltpu.make_async_copy(o_v.at[b0], out_slice(NCH - NBUF + b0), sos[b0]).wait()


def kernel(x, edge_index, W, b):
    xf = x.reshape(C, N).astype(jnp.float32)
    xp = jnp.pad(xf, ((0, 0), (0, N_PAD - N)))
    wd = (W[:, :C] - W[:, C:]).astype(jnp.float32)
    w2 = W[:, C:].astype(jnp.float32)
    hb = jnp.broadcast_to(0.5 * b.astype(jnp.float32), (8, O))

    ei = edge_index.astype(jnp.int32)
    i1 = jnp.pad(ei[1, 0], ((0, N_PAD - N), (0, 0))).reshape(NW, NCH, CH * K)
    i0 = jnp.pad(ei[0, 0], ((0, N_PAD - N), (0, 0))).reshape(NW, NCH, CH * K)

    y1, y2 = _build_tables(xp, wd, w2, hb)
    y1i = lax.bitcast_convert_type(y1.reshape(N_PAD, OW, 2), jnp.int32)
    y2i = lax.bitcast_convert_type(y2.reshape(N_PAD, OW, 2), jnp.int32)
    out_i = _sc_gather_max(y1i, y2i, i1, i0)
    out_rows = lax.bitcast_convert_type(out_i, jnp.bfloat16).reshape(N_PAD, O)
    return out_rows[:N].astype(jnp.float32).T.reshape(1, O, N, 1)


# speedup vs baseline: 1.2464x; 1.2464x over previous
import functools

import jax
import jax.numpy as jnp
from jax import lax
from jax.experimental import pallas as pl
from jax.experimental.pallas import tpu as pltpu
from jax.experimental.pallas import tpu_sc as plsc

C = 128
O = 128
OW = O // 2
WPT = 4
N = 10000
K = 16
L = 16

NC, NS = 2, 16
N_PAD = 10240
NSC = N_PAD // NC
CHN = 256
NCHU = NSC // CHN
BN = 2560


def _mm_body(x_ref, wd_ref, w2_ref, hb_ref, z1_ref, z2_ref):
    xb = x_ref[...]
    hb = hb_ref[:, 0:1]
    dn = (((1,), (0,)), ((), ()))
    z1_ref[...] = (lax.dot_general(wd_ref[...], xb, dn,
                                   preferred_element_type=jnp.float32)
                   + hb).astype(jnp.bfloat16)
    z2_ref[...] = (lax.dot_general(w2_ref[...], xb, dn,
                                   preferred_element_type=jnp.float32)
                   + hb).astype(jnp.bfloat16)


def _build_tables(xp, wd, w2, hb):
    nb = N_PAD // BN
    return pl.pallas_call(
        _mm_body,
        grid=(nb,),
        in_specs=[
            pl.BlockSpec((C, BN), lambda i: (0, i)),
            pl.BlockSpec((O, C), lambda i: (0, 0)),
            pl.BlockSpec((O, C), lambda i: (0, 0)),
            pl.BlockSpec((O, 128), lambda i: (0, 0)),
        ],
        out_specs=[
            pl.BlockSpec((O, BN), lambda i: (0, i)),
            pl.BlockSpec((O, BN), lambda i: (0, i)),
        ],
        out_shape=[
            jax.ShapeDtypeStruct((O, N_PAD), jnp.bfloat16),
            jax.ShapeDtypeStruct((O, N_PAD), jnp.bfloat16),
        ],
    )(xp, wd, w2, hb)


@functools.partial(
    pl.kernel,
    mesh=plsc.VectorSubcoreMesh(core_axis_name="c", subcore_axis_name="s"),
    out_type=jax.ShapeDtypeStruct((OW, N_PAD), jnp.int32),
    compiler_params=pltpu.CompilerParams(use_tc_tiling_on_sc=False,
                                         needs_layout_passes=False),
    scratch_types=[
        pltpu.VMEM((WPT, N_PAD), jnp.int32),
        pltpu.VMEM((WPT, N_PAD), jnp.int32),
        pltpu.VMEM((2, K, CHN), jnp.int32),
        pltpu.VMEM((2, K, CHN), jnp.int32),
        pltpu.VMEM((2, WPT, CHN), jnp.int32),
        pltpu.SemaphoreType.DMA,
        pltpu.SemaphoreType.DMA,
        pltpu.SemaphoreType.DMA,
        pltpu.SemaphoreType.DMA,
    ],
)
def _sc_gather_max(y1t_hbm, y2t_hbm, i1_hbm, i0_hbm, out_hbm,
                   tbl1, tbl2, i1_v, i0_v, o_v, si0, si1, so0, so1):
    cid = lax.axis_index("c")
    sid = lax.axis_index("s")
    sis = (si0, si1)
    sos = (so0, so1)
    wrows = pl.ds(sid * WPT, WPT)

    pltpu.sync_copy(y1t_hbm.at[wrows], tbl1)
    pltpu.sync_copy(y2t_hbm.at[wrows], tbl2)

    wvecs = [jnp.full((L,), w, jnp.int32) for w in range(WPT)]

    def issue_idx(ch, b):
        pltpu.async_copy(i1_hbm.at[cid, ch], i1_v.at[b], sis[b])
        pltpu.async_copy(i0_hbm.at[cid, ch], i0_v.at[b], sis[b])

    def wait_idx(ch, b):
        pltpu.make_async_copy(i1_hbm.at[cid, ch], i1_v.at[b], sis[b]).wait()
        pltpu.make_async_copy(i0_hbm.at[cid, ch], i0_v.at[b], sis[b]).wait()

    def out_slice(ch):
        return out_hbm.at[wrows, pl.ds(cid * NSC + ch * CHN, CHN)]

    def compute(b):
        def group_body(g, carry):
            sl = pl.ds(g * L, L)
            acc = [None] * WPT
            for k in range(K):
                idx1 = i1_v[b, k, sl]
                idx0 = i0_v[b, k, sl]
                for w in range(WPT):
                    s = (plsc.bitcast(plsc.load_gather(tbl1, [wvecs[w], idx1]),
                                      jnp.bfloat16)
                         + plsc.bitcast(plsc.load_gather(tbl2, [wvecs[w], idx0]),
                                        jnp.bfloat16))
                    acc[w] = s if k == 0 else jnp.maximum(acc[w], s)
            for w in range(WPT):
                o_v[b, w, sl] = plsc.bitcast(acc[w], jnp.int32)
            return carry

        lax.fori_loop(0, CHN // L, group_body, 0, unroll=False)

    issue_idx(0, 0)
    issue_idx(1, 1)

    def pair_body(cp, carry):
        for b in range(2):
            ch = 2 * cp + b
            wait_idx(ch, b)

            @pl.when(cp > 0)
            def _():
                pltpu.make_async_copy(o_v.at[b], out_slice(ch - 2), sos[b]).wait()

            compute(b)
            pltpu.async_copy(o_v.at[b], out_slice(ch), sos[b])

            @pl.when(cp < NCHU // 2 - 1)
            def _():
                issue_idx(ch + 2, b)
        return carry

    lax.fori_loop(0, NCHU // 2, pair_body, 0, unroll=False)
    pltpu.make_async_copy(o_v.at[0], out_slice(NCHU - 2), sos[0]).wait()
    pltpu.make_async_copy(o_v.at[1], out_slice(NCHU - 1), sos[1]).wait()


def kernel(x, edge_index, W, b):
    xf = x.reshape(C, N).astype(jnp.float32)
    xp = jnp.pad(xf, ((0, 0), (0, N_PAD - N)))
    wd = (W[:, :C] - W[:, C:]).astype(jnp.float32)
    w2 = W[:, C:].astype(jnp.float32)
    hb = jnp.broadcast_to(0.5 * b.astype(jnp.float32)[:, None], (O, 128))

    ei = edge_index.astype(jnp.int32)

    def prep_idx(a):
        ap = jnp.pad(a, ((0, N_PAD - N), (0, 0)))
        return ap.T.reshape(K, NC, NCHU, CHN).transpose(1, 2, 0, 3)

    i1 = prep_idx(ei[1, 0])
    i0 = prep_idx(ei[0, 0])

    z1, z2 = _build_tables(xp, wd, w2, hb)

    def pack(z):
        return lax.bitcast_convert_type(
            z.reshape(OW, 2, N_PAD).transpose(0, 2, 1), jnp.int32)

    out_t = _sc_gather_max(pack(z1), pack(z2), i1, i0)
    out_bf = lax.bitcast_convert_type(out_t, jnp.bfloat16)
    out_full = out_bf.transpose(0, 2, 1).reshape(O, N_PAD)
    return out_full[:, :N].astype(jnp.float32).reshape(1, O, N, 1)


# speedup vs baseline: 1.3119x; 1.0525x over previous
import functools

import jax
import jax.numpy as jnp
from jax import lax
from jax.experimental import pallas as pl
from jax.experimental.pallas import tpu as pltpu
from jax.experimental.pallas import tpu_sc as plsc

C = 128
O = 128
OW = O // 2
WPT = 4
N = 10000
K = 16
L = 16

NC, NS = 2, 16
N_PAD = 10240
NSC = N_PAD // NC
CHN = 256
NCHU = NSC // CHN
BN = 2560
BN2 = 2560


def _mm_body(x_ref, wd_ref, w2_ref, hb_ref, t1_ref, t2_ref):
    xb = x_ref[...]
    hb = hb_ref[:, 0:1]
    dn = (((1,), (0,)), ((), ()))

    def pack(wmat):
        z = (lax.dot_general(wmat, xb, dn,
                             preferred_element_type=jnp.float32)
             + hb).astype(jnp.bfloat16)
        lo = lax.bitcast_convert_type(z[:OW, :], jnp.uint16).astype(jnp.uint32)
        hi = lax.bitcast_convert_type(z[OW:, :], jnp.uint16).astype(jnp.uint32)
        return lax.bitcast_convert_type(lo | (hi << 16), jnp.int32)

    t1_ref[...] = pack(wd_ref[...])
    t2_ref[...] = pack(w2_ref[...])


def _build_tables(xp, wd, w2, hb):
    nb = N_PAD // BN
    return pl.pallas_call(
        _mm_body,
        grid=(nb,),
        in_specs=[
            pl.BlockSpec((C, BN), lambda i: (0, i)),
            pl.BlockSpec((O, C), lambda i: (0, 0)),
            pl.BlockSpec((O, C), lambda i: (0, 0)),
            pl.BlockSpec((O, 128), lambda i: (0, 0)),
        ],
        out_specs=[
            pl.BlockSpec((OW, BN), lambda i: (0, i)),
            pl.BlockSpec((OW, BN), lambda i: (0, i)),
        ],
        out_shape=[
            jax.ShapeDtypeStruct((OW, N_PAD), jnp.int32),
            jax.ShapeDtypeStruct((OW, N_PAD), jnp.int32),
        ],
    )(xp, wd, w2, hb)


def _unpack_body(t_ref, o_ref):
    u = lax.bitcast_convert_type(t_ref[...], jnp.uint32)
    lo = lax.bitcast_convert_type((u & 0xFFFF).astype(jnp.uint16),
                                  jnp.bfloat16).astype(jnp.float32)
    hi = lax.bitcast_convert_type((u >> 16).astype(jnp.uint16),
                                  jnp.bfloat16).astype(jnp.float32)
    o_ref[0:OW, :] = lo
    o_ref[OW:O, :] = hi


def _unpack(out_t):
    return pl.pallas_call(
        _unpack_body,
        grid=(N_PAD // BN2,),
        in_specs=[pl.BlockSpec((OW, BN2), lambda i: (0, i))],
        out_specs=pl.BlockSpec((O, BN2), lambda i: (0, i)),
        out_shape=jax.ShapeDtypeStruct((O, N_PAD), jnp.float32),
    )(out_t)


@functools.partial(
    pl.kernel,
    mesh=plsc.VectorSubcoreMesh(core_axis_name="c", subcore_axis_name="s"),
    out_type=jax.ShapeDtypeStruct((OW, N_PAD), jnp.int32),
    compiler_params=pltpu.CompilerParams(use_tc_tiling_on_sc=False,
                                         needs_layout_passes=False),
    scratch_types=[
        pltpu.VMEM((WPT, N_PAD), jnp.int32),
        pltpu.VMEM((WPT, N_PAD), jnp.int32),
        pltpu.VMEM((2, CHN * K), jnp.int32),
        pltpu.VMEM((2, CHN * K), jnp.int32),
        pltpu.VMEM((2, WPT, CHN), jnp.int32),
        pltpu.SemaphoreType.DMA,
        pltpu.SemaphoreType.DMA,
        pltpu.SemaphoreType.DMA,
        pltpu.SemaphoreType.DMA,
    ],
)
def _sc_gather_max(y1t_hbm, y2t_hbm, i1_hbm, i0_hbm, out_hbm,
                   tbl1, tbl2, i1_v, i0_v, o_v, si0, si1, so0, so1):
    cid = lax.axis_index("c")
    sid = lax.axis_index("s")
    sis = (si0, si1)
    sos = (so0, so1)
    wrows = pl.ds(sid * WPT, WPT)

    pltpu.sync_copy(y1t_hbm.at[wrows], tbl1)
    pltpu.sync_copy(y2t_hbm.at[wrows], tbl2)

    wvecs = [jnp.full((L,), w, jnp.int32) for w in range(WPT)]
    iv = jnp.arange(L, dtype=jnp.int32) * K

    def issue_idx(ch, b):
        pltpu.async_copy(i1_hbm.at[cid, ch], i1_v.at[b], sis[b])
        pltpu.async_copy(i0_hbm.at[cid, ch], i0_v.at[b], sis[b])

    def wait_idx(ch, b):
        pltpu.make_async_copy(i1_hbm.at[cid, ch], i1_v.at[b], sis[b]).wait()
        pltpu.make_async_copy(i0_hbm.at[cid, ch], i0_v.at[b], sis[b]).wait()

    def out_slice(ch):
        return out_hbm.at[wrows, pl.ds(cid * NSC + ch * CHN, CHN)]

    def compute(b):
        bvec = jnp.full((L,), b, jnp.int32)

        def group_body(g, carry):
            sl = pl.ds(g * L, L)
            acc = [None] * WPT
            for k in range(K):
                addr = iv + (g * (L * K) + k)
                idx1 = plsc.load_gather(i1_v, [bvec, addr])
                idx0 = plsc.load_gather(i0_v, [bvec, addr])
                for w in range(WPT):
                    s = (plsc.bitcast(plsc.load_gather(tbl1, [wvecs[w], idx1]),
                                      jnp.bfloat16)
                         + plsc.bitcast(plsc.load_gather(tbl2, [wvecs[w], idx0]),
                                        jnp.bfloat16))
                    acc[w] = s if k == 0 else jnp.maximum(acc[w], s)
            for w in range(WPT):
                o_v[b, w, sl] = plsc.bitcast(acc[w], jnp.int32)
            return carry

        lax.fori_loop(0, CHN // L, group_body, 0, unroll=False)

    issue_idx(0, 0)
    issue_idx(1, 1)

    def pair_body(cp, carry):
        for b in range(2):
            ch = 2 * cp + b
            wait_idx(ch, b)

            @pl.when(cp > 0)
            def _():
                pltpu.make_async_copy(o_v.at[b], out_slice(ch - 2), sos[b]).wait()

            compute(b)
            pltpu.async_copy(o_v.at[b], out_slice(ch), sos[b])

            @pl.when(cp < NCHU // 2 - 1)
            def _():
                issue_idx(ch + 2, b)
        return carry

    lax.fori_loop(0, NCHU // 2, pair_body, 0, unroll=False)
    pltpu.make_async_copy(o_v.at[0], out_slice(NCHU - 2), sos[0]).wait()
    pltpu.make_async_copy(o_v.at[1], out_slice(NCHU - 1), sos[1]).wait()


def kernel(x, edge_index, W, b):
    xf = x.reshape(C, N).astype(jnp.float32)
    xp = jnp.pad(xf, ((0, 0), (0, N_PAD - N)))
    wd = (W[:, :C] - W[:, C:]).astype(jnp.float32)
    w2 = W[:, C:].astype(jnp.float32)
    hb = jnp.broadcast_to(0.5 * b.astype(jnp.float32)[:, None], (O, 128))

    ei = edge_index.astype(jnp.int32)

    def prep_idx(a):
        return jnp.pad(a, ((0, N_PAD - N), (0, 0))).reshape(NC, NCHU, CHN * K)

    i1 = prep_idx(ei[1, 0])
    i0 = prep_idx(ei[0, 0])

    t1, t2 = _build_tables(xp, wd, w2, hb)
    out_t = _sc_gather_max(t1, t2, i1, i0)
    return _unpack(out_t)[:, :N].reshape(1, O, N, 1)


# speedup vs baseline: 1.7649x; 1.3452x over previous
import functools

import jax
import jax.numpy as jnp
from jax import lax
from jax.experimental import pallas as pl
from jax.experimental.pallas import tpu as pltpu
from jax.experimental.pallas import tpu_sc as plsc

C = 128
O = 128
OW = O // 2
WPT = 4
N = 10000
K = 16
L = 16

NC, NS = 2, 16
N_PAD = 10240
NSC = N_PAD // NC
CHN = 256
NCHU = NSC // CHN
BN = 2560
BN2 = 2560


def _mm_body(x_ref, wd_ref, w2_ref, hb_ref, t1_ref, t2_ref):
    xb = x_ref[...]
    hb = hb_ref[:, 0:1]
    dn = (((1,), (0,)), ((), ()))

    def pack(wmat):
        z = (lax.dot_general(wmat, xb, dn,
                             preferred_element_type=jnp.float32)
             + hb).astype(jnp.bfloat16)
        lo = lax.bitcast_convert_type(z[:OW, :], jnp.uint16).astype(jnp.uint32)
        hi = lax.bitcast_convert_type(z[OW:, :], jnp.uint16).astype(jnp.uint32)
        return lax.bitcast_convert_type(lo | (hi << 16), jnp.int32)

    t1_ref[...] = pack(wd_ref[...])
    t2_ref[...] = pack(w2_ref[...])


def _build_tables(xp, wd, w2, hb):
    return pl.pallas_call(
        _mm_body,
        grid=(N_PAD // BN,),
        in_specs=[
            pl.BlockSpec((C, BN), lambda i: (0, i)),
            pl.BlockSpec((O, C), lambda i: (0, 0)),
            pl.BlockSpec((O, C), lambda i: (0, 0)),
            pl.BlockSpec((O, 128), lambda i: (0, 0)),
        ],
        out_specs=[
            pl.BlockSpec((OW, BN), lambda i: (0, i)),
            pl.BlockSpec((OW, BN), lambda i: (0, i)),
        ],
        out_shape=[
            jax.ShapeDtypeStruct((OW, N_PAD), jnp.int32),
            jax.ShapeDtypeStruct((OW, N_PAD), jnp.int32),
        ],
    )(xp, wd, w2, hb)


def _unpack_body(t_ref, o_ref):
    u = lax.bitcast_convert_type(t_ref[...], jnp.uint32)
    lo = lax.bitcast_convert_type((u & 0xFFFF).astype(jnp.uint16),
                                  jnp.bfloat16).astype(jnp.float32)
    hi = lax.bitcast_convert_type((u >> 16).astype(jnp.uint16),
                                  jnp.bfloat16).astype(jnp.float32)
    o_ref[0:OW, :] = lo
    o_ref[OW:O, :] = hi


def _unpack(out_t):
    return pl.pallas_call(
        _unpack_body,
        grid=(N_PAD // BN2,),
        in_specs=[pl.BlockSpec((OW, BN2), lambda i: (0, i))],
        out_specs=pl.BlockSpec((O, BN2), lambda i: (0, i)),
        out_shape=jax.ShapeDtypeStruct((O, N_PAD), jnp.float32),
    )(out_t)


@functools.partial(
    pl.kernel,
    mesh=plsc.VectorSubcoreMesh(core_axis_name="c", subcore_axis_name="s"),
    out_type=jax.ShapeDtypeStruct((OW, N_PAD), jnp.int32),
    compiler_params=pltpu.CompilerParams(use_tc_tiling_on_sc=False,
                                         needs_layout_passes=False),
    scratch_types=[
        pltpu.VMEM((WPT, N_PAD), jnp.int32),
        pltpu.VMEM((WPT, N_PAD), jnp.int32),
        pltpu.VMEM((2, K, CHN), jnp.int32),
        pltpu.VMEM((2, K, CHN), jnp.int32),
        pltpu.VMEM((2, WPT, CHN), jnp.int32),
        pltpu.SemaphoreType.DMA,
        pltpu.SemaphoreType.DMA,
        pltpu.SemaphoreType.DMA,
        pltpu.SemaphoreType.DMA,
    ],
)
def _sc_gather_max(y1t_hbm, y2t_hbm, i1_hbm, i0_hbm, out_hbm,
                   tbl1, tbl2, i1_v, i0_v, o_v, si0, si1, so0, so1):
    cid = lax.axis_index("c")
    sid = lax.axis_index("s")
    sis = (si0, si1)
    sos = (so0, so1)
    wrows = pl.ds(sid * WPT, WPT)

    pltpu.sync_copy(y1t_hbm.at[wrows], tbl1)
    pltpu.sync_copy(y2t_hbm.at[wrows], tbl2)

    wvecs = [jnp.full((L,), w, jnp.int32) for w in range(WPT)]

    def issue_idx(ch, b):
        pltpu.async_copy(i1_hbm.at[cid, ch], i1_v.at[b], sis[b])
        pltpu.async_copy(i0_hbm.at[cid, ch], i0_v.at[b], sis[b])

    def wait_idx(ch, b):
        pltpu.make_async_copy(i1_hbm.at[cid, ch], i1_v.at[b], sis[b]).wait()
        pltpu.make_async_copy(i0_hbm.at[cid, ch], i0_v.at[b], sis[b]).wait()

    def out_slice(ch):
        return out_hbm.at[wrows, pl.ds(cid * NSC + ch * CHN, CHN)]

    def compute(b):
        def group_body(g, carry):
            sl = pl.ds(g * L, L)
            acc = [None] * WPT
            for k in range(K):
                idx1 = i1_v[b, k, sl]
                idx0 = i0_v[b, k, sl]
                for w in range(WPT):
                    s = (plsc.bitcast(plsc.load_gather(tbl1, [wvecs[w], idx1]),
                                      jnp.bfloat16)
                         + plsc.bitcast(plsc.load_gather(tbl2, [wvecs[w], idx0]),
                                        jnp.bfloat16))
                    acc[w] = s if k == 0 else jnp.maximum(acc[w], s)
            for w in range(WPT):
                o_v[b, w, sl] = plsc.bitcast(acc[w], jnp.int32)
            return carry

        lax.fori_loop(0, CHN // L, group_body, 0, unroll=False)

    issue_idx(0, 0)
    issue_idx(1, 1)

    def pair_body(cp, carry):
        for b in range(2):
            ch = 2 * cp + b
            wait_idx(ch, b)

            @pl.when(cp > 0)
            def _():
                pltpu.make_async_copy(o_v.at[b], out_slice(ch - 2), sos[b]).wait()

            compute(b)
            pltpu.async_copy(o_v.at[b], out_slice(ch), sos[b])

            @pl.when(cp < NCHU // 2 - 1)
            def _():
                issue_idx(ch + 2, b)
        return carry

    lax.fori_loop(0, NCHU // 2, pair_body, 0, unroll=False)
    pltpu.make_async_copy(o_v.at[0], out_slice(NCHU - 2), sos[0]).wait()
    pltpu.make_async_copy(o_v.at[1], out_slice(NCHU - 1), sos[1]).wait()


def kernel(x, edge_index, W, b):
    wd = (W[:, :C] - W[:, C:]).astype(jnp.float32)
    w2 = W[:, C:].astype(jnp.float32)
    hb = jnp.broadcast_to(0.5 * b.astype(jnp.float32)[:, None], (O, 128))

    ei = edge_index.astype(jnp.int32)

    def prep_idx(a):
        ap = jnp.pad(a, ((0, N_PAD - N), (0, 0)))
        return ap.T.reshape(K, NC, NCHU, CHN).transpose(1, 2, 0, 3)

    i1 = prep_idx(ei[1, 0])
    i0 = prep_idx(ei[0, 0])

    xp = jnp.pad(x.reshape(C, N).astype(jnp.float32), ((0, 0), (0, N_PAD - N)))
    t1, t2 = _build_tables(xp, wd, w2, hb)
    out_t = _sc_gather_max(t1, t2, i1, i0)
    return _unpack(out_t)[:, :N].reshape(1, O, N, 1)
